# Initial kernel scaffold; baseline (speedup 1.0000x reference)
import jax
import jax.numpy as jnp
from jax.experimental import pallas as pl


def kernel(source, targets, emb, W_h, b_h, W_out, b_out):
    raise NotImplementedError("write your pallas kernel here")


# speedup vs baseline: 1.4403x; 1.4403x over previous
import functools

import jax
import jax.numpy as jnp
from jax.experimental import pallas as pl
from jax.experimental.pallas import tpu as pltpu
from jax.experimental.pallas import tpu_sc as plsc


def _pick_tile(n: int) -> int:
    for t in (4000, 2048, 2000, 1600, 1280, 1024, 800, 512, 400, 256, 200, 128, 8):
        if n % t == 0 and t % 8 == 0:
            return t
    return n


def _sc_gathers(e128, w128, b128, i_src, i_tgt, i_b):
    B = i_src.shape[0]
    mesh = plsc.VectorSubcoreMesh(core_axis_name="c", subcore_axis_name="s")
    nw = 32
    bpw = B // nw

    @functools.partial(
        pl.kernel, mesh=mesh,
        out_type=(
            jax.ShapeDtypeStruct((B, 128), e128.dtype),
            jax.ShapeDtypeStruct((B, 128), w128.dtype),
            jax.ShapeDtypeStruct((B, 128), b128.dtype),
        ),
        scratch_types=[
            pltpu.VMEM((bpw,), jnp.int32),
            pltpu.VMEM((bpw,), jnp.int32),
            pltpu.VMEM((bpw,), jnp.int32),
            pltpu.VMEM((bpw, 128), jnp.float32),
            pltpu.VMEM((bpw, 128), jnp.float32),
            pltpu.VMEM((bpw, 128), jnp.float32),
            pltpu.SemaphoreType.DMA,
            pltpu.SemaphoreType.DMA,
            pltpu.SemaphoreType.DMA,
        ],
    )
    def k(e_hbm, w_hbm, b_hbm, is_hbm, it_hbm, ib_hbm,
          o1_hbm, o2_hbm, o3_hbm,
          i1_v, i2_v, i3_v, r1_v, r2_v, r3_v, s1, s2, s3):
        wid = jax.lax.axis_index("s") * 2 + jax.lax.axis_index("c")
        base = wid * bpw
        pltpu.sync_copy(is_hbm.at[pl.ds(base, bpw)], i1_v)
        pltpu.sync_copy(it_hbm.at[pl.ds(base, bpw)], i2_v)
        pltpu.sync_copy(ib_hbm.at[pl.ds(base, bpw)], i3_v)
        c1 = pltpu.async_copy(e_hbm.at[i1_v], r1_v, s1)
        c2 = pltpu.async_copy(w_hbm.at[i2_v], r2_v, s2)
        c3 = pltpu.async_copy(b_hbm.at[i3_v], r3_v, s3)
        c1.wait()
        c2.wait()
        c3.wait()
        pltpu.sync_copy(r1_v, o1_hbm.at[pl.ds(base, bpw)])
        pltpu.sync_copy(r2_v, o2_hbm.at[pl.ds(base, bpw)])
        pltpu.sync_copy(r3_v, o3_hbm.at[pl.ds(base, bpw)])

    return k(e128, w128, b128, i_src, i_tgt, i_b)


def _tc_body(g2_ref, sm_ref, wh_ref, bh_ref, wo_ref, bo_ref,
             w2_ref, tm_ref, b2_ref, tb_ref,
             lat_out, loss_out, lat_sc, acc_sc, *, nt, d):
    j = pl.program_id(0)

    @pl.when(j == 0)
    def _init():
        g2 = g2_ref[...]
        gsel = jnp.where(sm_ref[...] == 0, g2[:, :d], g2[:, d:])
        lat = jax.lax.dot_general(
            gsel, wh_ref[...],
            (((1,), (1,)), ((), ())), preferred_element_type=jnp.float32)
        lat = lat + bh_ref[...]
        nrm = jnp.sqrt(jnp.sum(lat * lat, axis=1, keepdims=True))
        den = jnp.where(nrm == 0.0, 1.0, nrm)
        lat = lat / den
        lat_sc[...] = lat
        lat_out[...] = lat
        acc_sc[...] = jnp.zeros_like(acc_sc)

    lat = lat_sc[...]
    logits = jax.lax.dot_general(
        lat.astype(jnp.bfloat16), wo_ref[...].astype(jnp.bfloat16),
        (((1,), (1,)), ((), ())), preferred_element_type=jnp.float32)
    logits = logits + bo_ref[0]
    acc_sc[...] += jnp.sum(jnp.exp(logits), axis=1, keepdims=True)

    @pl.when(j == nt - 1)
    def _fin():
        b = lat.shape[0]
        w2 = w2_ref[...]
        wsel = jnp.where(tm_ref[...] == 0, w2[:, :d], w2[:, d:])
        tgt_dot = jnp.sum(lat * wsel, axis=1)
        onehot = (jax.lax.broadcasted_iota(jnp.int32, (b, 128), 1)
                  == tb_ref[...])
        tgt_b = jnp.sum(jnp.where(onehot, b2_ref[...], 0.0), axis=1)
        lse = jnp.log(acc_sc[...][:, 0])
        loss_out[...] = jnp.mean(lse - tgt_dot - tgt_b).reshape(1, 1)


def kernel(source, targets, emb, W_h, b_h, W_out, b_out):
    B = source.shape[0]
    N, D = emb.shape
    L = W_h.shape[0]
    src = source.astype(jnp.int32)
    tgt = targets.astype(jnp.int32)

    e128 = emb.reshape(N // 2, 2 * D)
    w128 = W_out.reshape(N // 2, 2 * L)
    npad = (-N) % 128
    b128 = jnp.pad(b_out, (0, npad)).reshape((N + npad) // 128, 128)

    g2, w2, b2 = _sc_gathers(e128, w128, b128, src // 2, tgt // 2, tgt // 128)

    smod = (src % 2).reshape(B, 1)
    tmod = (tgt % 2).reshape(B, 1)
    tb = (tgt % 128).reshape(B, 1)

    tile = _pick_tile(N)
    nt = N // tile
    b3 = b_out.reshape(nt, 1, tile)

    grid_spec = pltpu.PrefetchScalarGridSpec(
        num_scalar_prefetch=0,
        grid=(nt,),
        in_specs=[
            pl.BlockSpec((B, 2 * D), lambda j: (0, 0)),
            pl.BlockSpec((B, 1), lambda j: (0, 0)),
            pl.BlockSpec((L, D), lambda j: (0, 0)),
            pl.BlockSpec((1, L), lambda j: (0, 0)),
            pl.BlockSpec((tile, L), lambda j: (j, 0)),
            pl.BlockSpec((1, 1, tile), lambda j: (j, 0, 0)),
            pl.BlockSpec((B, 2 * L), lambda j: (0, 0)),
            pl.BlockSpec((B, 1), lambda j: (0, 0)),
            pl.BlockSpec((B, 128), lambda j: (0, 0)),
            pl.BlockSpec((B, 1), lambda j: (0, 0)),
        ],
        out_specs=[
            pl.BlockSpec((B, L), lambda j: (0, 0)),
            pl.BlockSpec((1, 1), lambda j: (0, 0)),
        ],
        scratch_shapes=[
            pltpu.VMEM((B, L), jnp.float32),
            pltpu.VMEM((B, 1), jnp.float32),
        ],
    )

    latent, loss = pl.pallas_call(
        functools.partial(_tc_body, nt=nt, d=D),
        grid_spec=grid_spec,
        out_shape=[
            jax.ShapeDtypeStruct((B, L), jnp.float32),
            jax.ShapeDtypeStruct((1, 1), jnp.float32),
        ],
        compiler_params=pltpu.CompilerParams(
            dimension_semantics=("arbitrary",),
        ),
    )(g2, smod, W_h, b_h.reshape(1, L), W_out, b3, w2, tmod, b2, tb)

    return latent, loss.reshape(())


# speedup vs baseline: 1.4538x; 1.0094x over previous
import functools

import jax
import jax.numpy as jnp
from jax.experimental import pallas as pl
from jax.experimental.pallas import tpu as pltpu
from jax.experimental.pallas import tpu_sc as plsc


def _pick_tile(n: int) -> int:
    for t in (4000, 2048, 2000, 1600, 1280, 1024, 800, 512, 400, 256, 200, 128, 8):
        if n % t == 0 and t % 8 == 0:
            return t
    return n


def _sc_gathers(e128, w128, b128, i_src, i_tgt, i_b):
    B = i_src.shape[0]
    mesh = plsc.VectorSubcoreMesh(core_axis_name="c", subcore_axis_name="s")
    nw = 32
    bpw = B // nw

    @functools.partial(
        pl.kernel, mesh=mesh,
        out_type=(
            jax.ShapeDtypeStruct((B, 128), e128.dtype),
            jax.ShapeDtypeStruct((B, 128), w128.dtype),
            jax.ShapeDtypeStruct((B, 128), b128.dtype),
        ),
        scratch_types=[
            pltpu.VMEM((bpw,), jnp.int32),
            pltpu.VMEM((bpw,), jnp.int32),
            pltpu.VMEM((bpw,), jnp.int32),
            pltpu.VMEM((bpw, 128), jnp.float32),
            pltpu.VMEM((bpw, 128), jnp.float32),
            pltpu.VMEM((bpw, 128), jnp.float32),
            pltpu.SemaphoreType.DMA,
            pltpu.SemaphoreType.DMA,
            pltpu.SemaphoreType.DMA,
        ],
    )
    def k(e_hbm, w_hbm, b_hbm, is_hbm, it_hbm, ib_hbm,
          o1_hbm, o2_hbm, o3_hbm,
          i1_v, i2_v, i3_v, r1_v, r2_v, r3_v, s1, s2, s3):
        wid = jax.lax.axis_index("s") * 2 + jax.lax.axis_index("c")
        base = wid * bpw
        pltpu.sync_copy(is_hbm.at[pl.ds(base, bpw)], i1_v)
        pltpu.sync_copy(it_hbm.at[pl.ds(base, bpw)], i2_v)
        pltpu.sync_copy(ib_hbm.at[pl.ds(base, bpw)], i3_v)
        c1 = pltpu.async_copy(e_hbm.at[i1_v], r1_v, s1)
        c2 = pltpu.async_copy(w_hbm.at[i2_v], r2_v, s2)
        c3 = pltpu.async_copy(b_hbm.at[i3_v], r3_v, s3)
        c1.wait()
        c2.wait()
        c3.wait()
        pltpu.sync_copy(r1_v, o1_hbm.at[pl.ds(base, bpw)])
        pltpu.sync_copy(r2_v, o2_hbm.at[pl.ds(base, bpw)])
        pltpu.sync_copy(r3_v, o3_hbm.at[pl.ds(base, bpw)])

    return k(e128, w128, b128, i_src, i_tgt, i_b)


def _tc_body(g2_ref, src_ref, wh_ref, bh_ref, wo_ref, bo_ref,
             w2_ref, tgt_ref, b2_ref,
             lat_out, loss_out, lat_sc, acc_sc, *, nt, d, tile):
    j = pl.program_id(0)

    @pl.when(j == 0)
    def _init():
        g2 = g2_ref[...]
        smod = jnp.bitwise_and(src_ref[...], 1)
        gsel = jnp.where(smod == 0, g2[:, :d], g2[:, d:])
        lat = jax.lax.dot_general(
            gsel, wh_ref[...],
            (((1,), (1,)), ((), ())), preferred_element_type=jnp.float32)
        lat = lat + bh_ref[...]
        nrm = jnp.sqrt(jnp.sum(lat * lat, axis=1, keepdims=True))
        den = jnp.where(nrm == 0.0, 1.0, nrm)
        lat = lat / den
        lat_sc[...] = lat
        lat_out[...] = lat
        acc_sc[...] = jnp.zeros_like(acc_sc)

    lat = lat_sc[...]
    lat_bf = lat.astype(jnp.bfloat16)
    bo_bf = bo_ref[0].astype(jnp.bfloat16)
    sub = tile // 4
    nfull = sub // 128
    rem = sub - nfull * 128
    for c in range(4):
        wsub = wo_ref[c * sub:(c + 1) * sub, :].astype(jnp.bfloat16)
        lg = jax.lax.dot_general(
            lat_bf, wsub,
            (((1,), (1,)), ((), ())), preferred_element_type=jnp.float32)
        ex = jnp.exp(lg.astype(jnp.bfloat16) + bo_bf[:, c * sub:(c + 1) * sub])
        s = ex[:, :128]
        for k in range(1, nfull):
            s = s + ex[:, k * 128:(k + 1) * 128]
        if rem:
            lanes = jax.lax.broadcasted_iota(jnp.int32, ex[:, :128].shape, 1)
            s = s + jnp.where(lanes < 128 - rem, jnp.bfloat16(0.0),
                              ex[:, sub - 128:])
        acc_sc[...] += s.astype(jnp.float32)

    @pl.when(j == nt - 1)
    def _fin():
        b = lat.shape[0]
        w2 = w2_ref[...]
        tmod = jnp.bitwise_and(tgt_ref[...], 1)
        wsel = jnp.where(tmod == 0, w2[:, :d], w2[:, d:])
        tgt_dot = jnp.sum(lat * wsel, axis=1)
        onehot = (jax.lax.broadcasted_iota(jnp.int32, (b, 128), 1)
                  == jnp.bitwise_and(tgt_ref[...], 127))
        tgt_b = jnp.sum(jnp.where(onehot, b2_ref[...], 0.0), axis=1)
        lse = jnp.log(jnp.sum(acc_sc[...], axis=1))
        loss_out[...] = jnp.mean(lse - tgt_dot - tgt_b).reshape(1, 1)


def kernel(source, targets, emb, W_h, b_h, W_out, b_out):
    B = source.shape[0]
    N, D = emb.shape
    L = W_h.shape[0]
    src = source.astype(jnp.int32)
    tgt = targets.astype(jnp.int32)

    e128 = emb.reshape(N // 2, 2 * D)
    w128 = W_out.reshape(N // 2, 2 * L)
    npad = (-N) % 128
    b128 = jnp.pad(b_out, (0, npad)).reshape((N + npad) // 128, 128)

    g2, w2, b2 = _sc_gathers(e128, w128, b128, src // 2, tgt // 2, tgt // 128)

    tile = _pick_tile(N)
    nt = N // tile
    b3 = b_out.reshape(nt, 1, tile)

    grid_spec = pltpu.PrefetchScalarGridSpec(
        num_scalar_prefetch=0,
        grid=(nt,),
        in_specs=[
            pl.BlockSpec((B, 2 * D), lambda j: (0, 0)),
            pl.BlockSpec((B, 1), lambda j: (0, 0)),
            pl.BlockSpec((L, D), lambda j: (0, 0)),
            pl.BlockSpec((1, L), lambda j: (0, 0)),
            pl.BlockSpec((tile, L), lambda j: (j, 0)),
            pl.BlockSpec((1, 1, tile), lambda j: (j, 0, 0)),
            pl.BlockSpec((B, 2 * L), lambda j: (0, 0)),
            pl.BlockSpec((B, 1), lambda j: (0, 0)),
            pl.BlockSpec((B, 128), lambda j: (0, 0)),
        ],
        out_specs=[
            pl.BlockSpec((B, L), lambda j: (0, 0)),
            pl.BlockSpec((1, 1), lambda j: (0, 0)),
        ],
        scratch_shapes=[
            pltpu.VMEM((B, L), jnp.float32),
            pltpu.VMEM((B, 128), jnp.float32),
        ],
    )

    latent, loss = pl.pallas_call(
        functools.partial(_tc_body, nt=nt, d=D, tile=tile),
        grid_spec=grid_spec,
        out_shape=[
            jax.ShapeDtypeStruct((B, L), jnp.float32),
            jax.ShapeDtypeStruct((1, 1), jnp.float32),
        ],
        compiler_params=pltpu.CompilerParams(
            dimension_semantics=("arbitrary",),
        ),
    )(g2, src.reshape(B, 1), W_h, b_h.reshape(1, L), W_out, b3, w2,
      tgt.reshape(B, 1), b2)

    return latent, loss.reshape(())
